# Initial kernel scaffold; baseline (speedup 1.0000x reference)
import jax
import jax.numpy as jnp
from jax import lax
from jax.experimental import pallas as pl
from jax.experimental.pallas import tpu as pltpu
from jax.experimental.pallas import tpu_sc as plsc

N = 10000
E = 320000
DIM = 128
H = 8
HD = 16

C = 128
NCHUNKS = E // C
NCORES = 2
NSUB = 16
NW = NCORES * NSUB
CPW = (NCHUNKS + NW - 1) // NW

_MESH = plsc.VectorSubcoreMesh(
    core_axis_name="c", subcore_axis_name="s",
    num_cores=NCORES, num_subcores=NSUB)


_PROJ_BLK = 1000


def _proj_body(x_ref, wq, bq, wk, bk, wv, bv, q_out, k_out, v_out):
    xb = x_ref[...]
    dn = (((1,), (1,)), ((), ()))
    q_out[...] = lax.dot_general(xb, wq[...], dn,
                                 preferred_element_type=jnp.float32) + bq[...]
    k_out[...] = lax.dot_general(xb, wk[...], dn,
                                 preferred_element_type=jnp.float32) + bk[...]
    v_out[...] = lax.dot_general(xb, wv[...], dn,
                                 preferred_element_type=jnp.float32) + bv[...]


def _project(x, Wq, bq, Wk, bk, Wv, bv):
    full = pl.BlockSpec((DIM, DIM), lambda i: (0, 0))
    brow = pl.BlockSpec((1, DIM), lambda i: (0, 0))
    blk = pl.BlockSpec((_PROJ_BLK, DIM), lambda i: (i, 0))
    out_sds = jax.ShapeDtypeStruct((N, DIM), jnp.float32)
    return pl.pallas_call(
        _proj_body,
        grid=(N // _PROJ_BLK,),
        in_specs=[blk, full, brow, full, brow, full, brow],
        out_specs=[blk, blk, blk],
        out_shape=[out_sds, out_sds, out_sds],
    )(x, Wq, bq.reshape(1, DIM), Wk, bk.reshape(1, DIM), Wv, bv.reshape(1, DIM))


def _scores_body(k_hbm, q_hbm, ei_hbm, zero8_hbm, score_hbm, sumpart_hbm,
                 src_idx, dst_idx, krows, qrows, scores_v, sum_acc,
                 sem0, sem1):
    cid = lax.axis_index("c")
    sid = lax.axis_index("s")
    wid = cid * NSUB + sid

    @pl.when(sid == 0)
    def _zero():
        pltpu.sync_copy(zero8_hbm, sum_acc)

    plsc.subcore_barrier()

    def chunk_body(i, carry):
        chunk = i * NW + wid

        @pl.when(chunk < NCHUNKS)
        def _():
            base = chunk * C
            pltpu.sync_copy(ei_hbm.at[0, pl.ds(base, C)], src_idx)
            pltpu.sync_copy(ei_hbm.at[1, pl.ds(base, C)], dst_idx)
            cp_k = pltpu.async_copy(k_hbm.at[src_idx], krows, sem0)
            cp_q = pltpu.async_copy(q_hbm.at[dst_idx], qrows, sem1)
            cp_k.wait()
            cp_q.wait()

            def edge_body(e, c2):
                for h in range(H):
                    kv = krows[e, pl.ds(h * HD, HD)]
                    qv = qrows[e, pl.ds(h * HD, HD)]
                    scores_v[e, h] = jnp.sum(kv * qv)
                return c2

            lax.fori_loop(0, C, edge_body, 0)
            pltpu.sync_copy(scores_v, score_hbm.at[pl.ds(base, C)])
            pltpu.sync_copy(scores_v, sum_acc.at[dst_idx], add=True)

        return carry

    lax.fori_loop(0, CPW, chunk_body, 0)
    plsc.subcore_barrier()

    @pl.when(sid == 0)
    def _out():
        pltpu.sync_copy(sum_acc, sumpart_hbm.at[cid])


_scores_call = pl.kernel(
    _scores_body,
    out_type=(jax.ShapeDtypeStruct((E, H), jnp.float32),
              jax.ShapeDtypeStruct((NCORES, N, H), jnp.float32)),
    mesh=_MESH,
    scratch_types=[
        pltpu.VMEM((C,), jnp.int32),
        pltpu.VMEM((C,), jnp.int32),
        pltpu.VMEM((C, DIM), jnp.float32),
        pltpu.VMEM((C, DIM), jnp.float32),
        pltpu.VMEM((C, H), jnp.float32),
        pltpu.VMEM_SHARED((N, H), jnp.float32),
        pltpu.SemaphoreType.DMA,
        pltpu.SemaphoreType.DMA,
    ],
)


def _inv_body(p_ref, inv_ref):
    inv_ref[...] = 1.0 / (p_ref[0] + p_ref[1])


def _inv_sum(sumpart):
    return pl.pallas_call(
        _inv_body,
        out_shape=jax.ShapeDtypeStruct((N, H), jnp.float32),
    )(sumpart)


def _agg_body(v_hbm, ei_hbm, score_hbm, inv_hbm, zero128_hbm, outpart_hbm,
              src_idx, dst_idx, vrows, scores_v, inv_t, out_acc, sem0):
    cid = lax.axis_index("c")
    sid = lax.axis_index("s")
    wid = cid * NSUB + sid

    @pl.when(sid == 0)
    def _zero():
        pltpu.sync_copy(zero128_hbm, out_acc)

    pltpu.sync_copy(inv_hbm, inv_t)
    plsc.subcore_barrier()

    def chunk_body(i, carry):
        chunk = i * NW + wid

        @pl.when(chunk < NCHUNKS)
        def _():
            base = chunk * C
            pltpu.sync_copy(ei_hbm.at[0, pl.ds(base, C)], src_idx)
            pltpu.sync_copy(ei_hbm.at[1, pl.ds(base, C)], dst_idx)
            cp_v = pltpu.async_copy(v_hbm.at[src_idx], vrows, sem0)
            pltpu.sync_copy(score_hbm.at[pl.ds(base, C)], scores_v)
            cp_v.wait()

            def edge_body(e, c2):
                d = dst_idx[e]
                for h in range(H):
                    w = scores_v[e, h] * inv_t[d, h]
                    vrows[e, pl.ds(h * HD, HD)] = vrows[e, pl.ds(h * HD, HD)] * w
                return c2

            lax.fori_loop(0, C, edge_body, 0)
            pltpu.sync_copy(vrows, out_acc.at[dst_idx], add=True)

        return carry

    lax.fori_loop(0, CPW, chunk_body, 0)
    plsc.subcore_barrier()

    @pl.when(sid == 0)
    def _out():
        pltpu.sync_copy(out_acc, outpart_hbm.at[cid])


_agg_call = pl.kernel(
    _agg_body,
    out_type=jax.ShapeDtypeStruct((NCORES, N, DIM), jnp.float32),
    mesh=_MESH,
    scratch_types=[
        pltpu.VMEM((C,), jnp.int32),
        pltpu.VMEM((C,), jnp.int32),
        pltpu.VMEM((C, DIM), jnp.float32),
        pltpu.VMEM((C, H), jnp.float32),
        pltpu.VMEM((N, H), jnp.float32),
        pltpu.VMEM_SHARED((N, DIM), jnp.float32),
        pltpu.SemaphoreType.DMA,
    ],
)


def _comb_body(p_ref, o_ref):
    o_ref[...] = p_ref[0] + p_ref[1]


def _combine(outpart):
    return pl.pallas_call(
        _comb_body,
        out_shape=jax.ShapeDtypeStruct((N, DIM), jnp.float32),
    )(outpart)


def kernel(x, edge_index, Wq, bq, Wk, bk, Wv, bv):
    Q, K, V = _project(x, Wq, bq, Wk, bk, Wv, bv)
    zeros8 = jnp.zeros((N, H), jnp.float32)
    zeros128 = jnp.zeros((N, DIM), jnp.float32)
    score, sumpart = _scores_call(K, Q, edge_index, zeros8)
    inv = _inv_sum(sumpart)
    outpart = _agg_call(V, edge_index, score, inv, zeros128)
    return _combine(outpart)


# speedup vs baseline: 32.5976x; 32.5976x over previous
import jax
import jax.numpy as jnp
from jax import lax
from jax.experimental import pallas as pl
from jax.experimental.pallas import tpu as pltpu
from jax.experimental.pallas import tpu_sc as plsc

N = 10000
E = 320000
DIM = 128
HDIM = 64
H = 8
HH = 4
HD = 16

C = 128
NCHUNKS = E // C
NCORES = 2
NSUB = 16
CPT = (NCHUNKS + NSUB - 1) // NSUB

_MESH = plsc.VectorSubcoreMesh(
    core_axis_name="c", subcore_axis_name="s",
    num_cores=NCORES, num_subcores=NSUB)

_SC_PARAMS = pltpu.CompilerParams(
    use_tc_tiling_on_sc=False, needs_layout_passes=False)


_PROJ_BLK = 1000


def _proj_body(x_ref, wq, bq, wk, bk, wv, bv,
               qlo, qhi, klo, khi, vlo, vhi):
    xb = x_ref[...]
    dn = (((1,), (1,)), ((), ()))
    q = lax.dot_general(xb, wq[...], dn,
                        preferred_element_type=jnp.float32) + bq[...]
    k = lax.dot_general(xb, wk[...], dn,
                        preferred_element_type=jnp.float32) + bk[...]
    v = lax.dot_general(xb, wv[...], dn,
                        preferred_element_type=jnp.float32) + bv[...]
    qlo[...] = q[:, :HDIM]
    qhi[...] = q[:, HDIM:]
    klo[...] = k[:, :HDIM]
    khi[...] = k[:, HDIM:]
    vlo[...] = v[:, :HDIM]
    vhi[...] = v[:, HDIM:]


def _project(x, Wq, bq, Wk, bk, Wv, bv):
    full = pl.BlockSpec((DIM, DIM), lambda i: (0, 0))
    brow = pl.BlockSpec((1, DIM), lambda i: (0, 0))
    blk = pl.BlockSpec((_PROJ_BLK, DIM), lambda i: (i, 0))
    hblk = pl.BlockSpec((_PROJ_BLK, HDIM), lambda i: (i, 0))
    hsds = jax.ShapeDtypeStruct((N, HDIM), jnp.float32)
    return pl.pallas_call(
        _proj_body,
        grid=(N // _PROJ_BLK,),
        in_specs=[blk, full, brow, full, brow, full, brow],
        out_specs=[hblk] * 6,
        out_shape=[hsds] * 6,
    )(x, Wq, bq.reshape(1, DIM), Wk, bk.reshape(1, DIM), Wv, bv.reshape(1, DIM))


def _scores_body(klo_hbm, khi_hbm, qlo_hbm, qhi_hbm, ei_hbm, zero4_hbm,
                 score_hbm, sum_hbm,
                 src_idx, dst_idx, krows, qrows, scores_v, sum_acc,
                 sem0, sem1):
    cid = lax.axis_index("c")
    sid = lax.axis_index("s")

    @pl.when(sid == 0)
    def _zero():
        pltpu.sync_copy(zero4_hbm, sum_acc)

    plsc.subcore_barrier()

    lanes = lax.iota(jnp.int32, HD)
    row_off = lanes >> 2
    col_idx = lanes & (HH - 1)
    lane_masks = [lanes == j for j in range(HD)]

    def chunk_body(i, carry):
        chunk = i * NSUB + sid

        @pl.when(chunk < NCHUNKS)
        def _():
            base = chunk * C
            pltpu.sync_copy(ei_hbm.at[0, pl.ds(base, C)], src_idx)
            pltpu.sync_copy(ei_hbm.at[1, pl.ds(base, C)], dst_idx)

            @pl.when(cid == 0)
            def _gather_lo():
                cp_k = pltpu.async_copy(klo_hbm.at[src_idx], krows, sem0)
                cp_q = pltpu.async_copy(qlo_hbm.at[dst_idx], qrows, sem1)
                cp_k.wait()
                cp_q.wait()

            @pl.when(cid == 1)
            def _gather_hi():
                cp_k = pltpu.async_copy(khi_hbm.at[src_idx], krows, sem0)
                cp_q = pltpu.async_copy(qhi_hbm.at[dst_idx], qrows, sem1)
                cp_k.wait()
                cp_q.wait()

            def quad_body(qd, c2):
                e0 = qd * 4
                res = jnp.zeros((HD,), jnp.float32)
                for ej in range(4):
                    for h in range(HH):
                        kv = krows[e0 + ej, pl.ds(h * HD, HD)]
                        qv = qrows[e0 + ej, pl.ds(h * HD, HD)]
                        tot = jnp.sum(kv * qv)
                        res = jnp.where(lane_masks[ej * HH + h],
                                        jnp.full((HD,), tot, jnp.float32),
                                        res)
                plsc.store_scatter(scores_v.at[pl.ds(e0, 4)],
                                   [row_off, col_idx], res)
                return c2

            lax.fori_loop(0, C // 4, quad_body, 0)
            pltpu.sync_copy(scores_v, score_hbm.at[cid, pl.ds(base, C)])
            pltpu.sync_copy(scores_v, sum_acc.at[dst_idx], add=True)

        return carry

    lax.fori_loop(0, CPT, chunk_body, 0)
    plsc.subcore_barrier()

    @pl.when(sid == 0)
    def _out():
        pltpu.sync_copy(sum_acc, sum_hbm.at[cid])


_scores_call = pl.kernel(
    _scores_body,
    out_type=(jax.ShapeDtypeStruct((NCORES, E, HH), jnp.float32),
              jax.ShapeDtypeStruct((NCORES, N, HH), jnp.float32)),
    mesh=_MESH,
    compiler_params=_SC_PARAMS,
    scratch_types=[
        pltpu.VMEM((C,), jnp.int32),
        pltpu.VMEM((C,), jnp.int32),
        pltpu.VMEM((C, HDIM), jnp.float32),
        pltpu.VMEM((C, HDIM), jnp.float32),
        pltpu.VMEM((C, HH), jnp.float32),
        pltpu.VMEM_SHARED((N, HH), jnp.float32),
        pltpu.SemaphoreType.DMA,
        pltpu.SemaphoreType.DMA,
    ],
)


def _inv_body(p_ref, inv_ref):
    inv_ref[...] = 1.0 / p_ref[...]


def _inv_sum(sums):
    return pl.pallas_call(
        _inv_body,
        out_shape=jax.ShapeDtypeStruct((NCORES, N, HH), jnp.float32),
    )(sums)


def _agg_body(vlo_hbm, vhi_hbm, ei_hbm, score_hbm, inv_hbm, zero64_hbm,
              outpart_hbm,
              src_idx, dst_idx, vrows, scores_v, inv_t, out_acc, sem0):
    cid = lax.axis_index("c")
    sid = lax.axis_index("s")

    @pl.when(sid == 0)
    def _zero():
        pltpu.sync_copy(zero64_hbm, out_acc)

    pltpu.sync_copy(inv_hbm.at[cid], inv_t)
    plsc.subcore_barrier()

    lanes = lax.iota(jnp.int32, HD)
    row_off = lanes >> 2
    col_idx = lanes & (HH - 1)

    def chunk_body(i, carry):
        chunk = i * NSUB + sid

        @pl.when(chunk < NCHUNKS)
        def _():
            base = chunk * C
            pltpu.sync_copy(ei_hbm.at[0, pl.ds(base, C)], src_idx)
            pltpu.sync_copy(ei_hbm.at[1, pl.ds(base, C)], dst_idx)

            @pl.when(cid == 0)
            def _gather_lo():
                pltpu.async_copy(vlo_hbm.at[src_idx], vrows, sem0).wait()

            @pl.when(cid == 1)
            def _gather_hi():
                pltpu.async_copy(vhi_hbm.at[src_idx], vrows, sem0).wait()

            pltpu.sync_copy(score_hbm.at[cid, pl.ds(base, C)], scores_v)

            def group_body(g, c2):
                dvec = dst_idx[pl.ds(g * HD, HD)]
                for jq in range(HD // 4):
                    e0 = g * HD + 4 * jq
                    rowi = jnp.zeros((HD,), jnp.int32)
                    for ej in range(4):
                        mask = (lanes >= ej * HH) & (lanes < (ej + 1) * HH)
                        rowi = jnp.where(
                            mask,
                            jnp.full((HD,), dvec[4 * jq + ej], jnp.int32),
                            rowi)
                    rowv = jnp.full((HD,), e0, jnp.int32) + row_off
                    sv = plsc.load_gather(scores_v, [rowv, col_idx])
                    iv = plsc.load_gather(inv_t, [rowi, col_idx])
                    w = sv * iv
                    for ej in range(4):
                        for h in range(HH):
                            ws = jnp.full((HD,), w[ej * HH + h], jnp.float32)
                            vrows[e0 + ej, pl.ds(h * HD, HD)] = (
                                vrows[e0 + ej, pl.ds(h * HD, HD)] * ws)
                return c2

            lax.fori_loop(0, C // HD, group_body, 0)
            pltpu.sync_copy(vrows, out_acc.at[dst_idx], add=True)

        return carry

    lax.fori_loop(0, CPT, chunk_body, 0)
    plsc.subcore_barrier()

    @pl.when(sid == 0)
    def _out():
        pltpu.sync_copy(out_acc, outpart_hbm.at[cid])


_agg_call = pl.kernel(
    _agg_body,
    out_type=jax.ShapeDtypeStruct((NCORES, N, HDIM), jnp.float32),
    mesh=_MESH,
    compiler_params=_SC_PARAMS,
    scratch_types=[
        pltpu.VMEM((C,), jnp.int32),
        pltpu.VMEM((C,), jnp.int32),
        pltpu.VMEM((C, HDIM), jnp.float32),
        pltpu.VMEM((C, HH), jnp.float32),
        pltpu.VMEM((N, HH), jnp.float32),
        pltpu.VMEM_SHARED((N, HDIM), jnp.float32),
        pltpu.SemaphoreType.DMA,
    ],
)


def _comb_body(p_ref, o_ref):
    o_ref[...] = jnp.concatenate([p_ref[0], p_ref[1]], axis=1)


def _combine(outpart):
    return pl.pallas_call(
        _comb_body,
        out_shape=jax.ShapeDtypeStruct((N, DIM), jnp.float32),
    )(outpart)


def kernel(x, edge_index, Wq, bq, Wk, bk, Wv, bv):
    qlo, qhi, klo, khi, vlo, vhi = _project(x, Wq, bq, Wk, bk, Wv, bv)
    zeros4 = jnp.zeros((N, HH), jnp.float32)
    zeros64 = jnp.zeros((N, HDIM), jnp.float32)
    score, sums = _scores_call(klo, khi, qlo, qhi, edge_index, zeros4)
    inv = _inv_sum(sums)
    outpart = _agg_call(vlo, vhi, edge_index, score, inv, zeros64)
    return _combine(outpart)
